# baseline (speedup 1.0000x reference)
import functools

import jax
import jax.numpy as jnp
from jax import lax
from jax.experimental import pallas as pl
from jax.experimental.pallas import tpu as pltpu
from jax.experimental.pallas import tpu_sc as plsc

H = 768
I = 3072
E = 64
N = 4096
BT = 256
NBLK = N // BT + E
NPAD = NBLK * BT
NW = 32
LB_COEF = 0.01


def _gate_body(x_ref, wg_ref, bg_ref, slot_ref, be_ref, na_ref, lb_ref):
    x = x_ref[...]
    wg = wg_ref[...]
    logits = lax.dot_general(
        x.astype(jnp.bfloat16), wg.astype(jnp.bfloat16), (((1,), (1,)), ((), ())),
        preferred_element_type=jnp.float32,
    ) + bg_ref[...]
    m = jnp.max(logits, axis=1, keepdims=True)
    col = lax.broadcasted_iota(jnp.int32, logits.shape, 1)
    eid = jnp.min(jnp.where(logits == m, col, E), axis=1, keepdims=True)
    oh = (col == eid).astype(jnp.int32)
    csum = oh
    sh = 1
    while sh < N:
        csum = csum + jnp.concatenate(
            [jnp.zeros((sh, E), jnp.int32), csum[: N - sh]], axis=0
        )
        sh *= 2
    counts = csum[N - 1 : N, :]
    rank = jnp.sum(oh * csum, axis=1, keepdims=True) - 1

    nb = (counts + BT - 1) // BT
    pb = nb * BT

    def lane_scan(v):
        s = 1
        while s < E:
            v = v + jnp.concatenate(
                [jnp.zeros((1, s), jnp.int32), v[:, : E - s]], axis=1
            )
            s *= 2
        return v

    cnb = lane_scan(nb)
    pstart = lane_scan(pb) - pb
    slot_ref[...] = jnp.sum(oh * pstart, axis=1, keepdims=True) + rank
    na = cnb[:, E - 1 : E]
    na_ref[...] = na
    rowb = lax.broadcasted_iota(jnp.int32, (NBLK, E), 0)
    ber = jnp.sum(
        (jnp.broadcast_to(cnb, (NBLK, E)) <= rowb).astype(jnp.int32),
        axis=1, keepdims=True,
    )
    bidx = lax.broadcasted_iota(jnp.int32, (NBLK, 1), 0)
    last_e = jnp.sum(jnp.where(bidx == na - 1, ber, 0), axis=0, keepdims=True)
    be_ref[...] = jnp.where(bidx < na, ber, last_e)
    mean = jnp.mean(logits, axis=0, keepdims=True)
    c = logits - mean
    var = jnp.sum(c * c, axis=0, keepdims=True) / (N - 1)
    ratio = var / (mean * mean + 1e-8)
    lb_ref[...] = LB_COEF * jnp.mean(ratio, axis=1, keepdims=True)


def _ffn_body(be_ref, na_ref, xs_ref, w1_ref, b1_ref, w2_ref, b2_ref, out_ref):
    b = pl.program_id(0)

    @pl.when(b < na_ref[0])
    def _():
        xb = xs_ref[...].astype(jnp.bfloat16)
        h = lax.dot_general(
            xb, w1_ref[0].astype(jnp.bfloat16), (((1,), (1,)), ((), ())),
            preferred_element_type=jnp.float32,
        ) + b1_ref[0]
        h = h * 0.5 * (1.0 + lax.erf(h * 0.7071067811865476))
        y = lax.dot_general(
            h.astype(jnp.bfloat16), w2_ref[0].astype(jnp.bfloat16),
            (((1,), (1,)), ((), ())),
            preferred_element_type=jnp.float32,
        )
        out_ref[...] = y + b2_ref[0]


def _sc_gather_rows(table, idx, n_rows, ch):
    rows_per_w = n_rows // NW
    chunks = rows_per_w // ch
    mesh = plsc.VectorSubcoreMesh(core_axis_name="c", subcore_axis_name="s")

    @functools.partial(
        pl.kernel,
        out_type=jax.ShapeDtypeStruct((n_rows, H), jnp.float32),
        scratch_types=[
            pltpu.VMEM((ch,), jnp.int32),
            pltpu.VMEM((ch, H), jnp.float32),
            pltpu.SemaphoreType.DMA,
        ],
        mesh=mesh,
    )
    def k(table_hbm, idx_hbm, out_hbm, idx_v, rows_v, sem):
        wid = lax.axis_index("s") * 2 + lax.axis_index("c")
        for c in range(chunks):
            base = wid * rows_per_w + c * ch
            pltpu.sync_copy(idx_hbm.at[pl.ds(base, ch)], idx_v)
            pltpu.async_copy(table_hbm.at[idx_v], rows_v, sem).wait()
            pltpu.sync_copy(rows_v, out_hbm.at[pl.ds(base, ch)])

    return k(table, idx)


def kernel(hidden_states, W1, b1, W2, b2, Wg, bg):
    Bs, Ss, Hd = hidden_states.shape
    x = hidden_states.reshape(-1, Hd)

    slot2, be2, na2, lb2 = pl.pallas_call(
        _gate_body,
        out_shape=(
            jax.ShapeDtypeStruct((N, 1), jnp.int32),
            jax.ShapeDtypeStruct((NBLK, 1), jnp.int32),
            jax.ShapeDtypeStruct((1, 1), jnp.int32),
            jax.ShapeDtypeStruct((1, 1), jnp.float32),
        ),
    )(x, Wg, bg.reshape(1, E))
    slot = slot2[:, 0]
    be = be2[:, 0]
    na = na2[0]

    ar_n = jnp.arange(N, dtype=jnp.int32)
    gidx = (jnp.arange(NPAD, dtype=jnp.int32) % N).at[slot].set(ar_n)
    pos = slot

    x_sorted = _sc_gather_rows(x, gidx, NPAD, 128)

    grid_spec = pltpu.PrefetchScalarGridSpec(
        num_scalar_prefetch=2,
        grid=(NBLK,),
        in_specs=[
            pl.BlockSpec((BT, H), lambda b, be, na: (b, 0)),
            pl.BlockSpec((1, I, H), lambda b, be, na: (be[b], 0, 0)),
            pl.BlockSpec((1, 1, I), lambda b, be, na: (be[b], 0, 0)),
            pl.BlockSpec((1, H, I), lambda b, be, na: (be[b], 0, 0)),
            pl.BlockSpec((1, 1, H), lambda b, be, na: (be[b], 0, 0)),
        ],
        out_specs=pl.BlockSpec((BT, H), lambda b, be, na: (b, 0)),
    )
    y_sorted = pl.pallas_call(
        _ffn_body,
        grid_spec=grid_spec,
        out_shape=jax.ShapeDtypeStruct((NPAD, H), jnp.float32),
    )(be, na, x_sorted, W1, b1.reshape(E, 1, I), W2, b2.reshape(E, 1, H))

    out = _sc_gather_rows(y_sorted, pos, N, 128)
    return out.reshape(Bs, Ss, Hd), lb2[0, 0]


# speedup vs baseline: 1.0663x; 1.0663x over previous
import functools

import jax
import jax.numpy as jnp
from jax import lax
from jax.experimental import pallas as pl
from jax.experimental.pallas import tpu as pltpu
from jax.experimental.pallas import tpu_sc as plsc

H = 768
I = 3072
E = 64
N = 4096
BT = 128
NBLK = N // BT + E
NPAD = NBLK * BT
NW = 32
LB_COEF = 0.01


def _gate_body(x_ref, wg_ref, bg_ref, slot_ref, be_ref, na_ref, lb_ref):
    x = x_ref[...]
    wg = wg_ref[...]
    logits = lax.dot_general(
        x.astype(jnp.bfloat16), wg.astype(jnp.bfloat16), (((1,), (1,)), ((), ())),
        preferred_element_type=jnp.float32,
    ) + bg_ref[...]
    m = jnp.max(logits, axis=1, keepdims=True)
    col = lax.broadcasted_iota(jnp.int32, logits.shape, 1)
    eid = jnp.min(jnp.where(logits == m, col, E), axis=1, keepdims=True)
    oh = (col == eid).astype(jnp.int32)
    csum = oh
    sh = 1
    while sh < N:
        csum = csum + jnp.concatenate(
            [jnp.zeros((sh, E), jnp.int32), csum[: N - sh]], axis=0
        )
        sh *= 2
    counts = csum[N - 1 : N, :]
    rank = jnp.sum(oh * csum, axis=1, keepdims=True) - 1

    nb = (counts + BT - 1) // BT
    pb = nb * BT

    def lane_scan(v):
        s = 1
        while s < E:
            v = v + jnp.concatenate(
                [jnp.zeros((1, s), jnp.int32), v[:, : E - s]], axis=1
            )
            s *= 2
        return v

    cnb = lane_scan(nb)
    pstart = lane_scan(pb) - pb
    slot_ref[...] = jnp.sum(oh * pstart, axis=1, keepdims=True) + rank
    na = cnb[:, E - 1 : E]
    na_ref[...] = na
    rowb = lax.broadcasted_iota(jnp.int32, (NBLK, E), 0)
    ber = jnp.sum(
        (jnp.broadcast_to(cnb, (NBLK, E)) <= rowb).astype(jnp.int32),
        axis=1, keepdims=True,
    )
    bidx = lax.broadcasted_iota(jnp.int32, (NBLK, 1), 0)
    last_e = jnp.sum(jnp.where(bidx == na - 1, ber, 0), axis=0, keepdims=True)
    be_ref[...] = jnp.where(bidx < na, ber, last_e)
    mean = jnp.mean(logits, axis=0, keepdims=True)
    c = logits - mean
    var = jnp.sum(c * c, axis=0, keepdims=True) / (N - 1)
    ratio = var / (mean * mean + 1e-8)
    lb_ref[...] = LB_COEF * jnp.mean(ratio, axis=1, keepdims=True)


def _ffn_body(be_ref, na_ref, xs_ref, w1_ref, b1_ref, w2_ref, b2_ref, out_ref):
    b = pl.program_id(0)

    @pl.when(b < na_ref[0])
    def _():
        xb = xs_ref[...].astype(jnp.bfloat16)
        h = lax.dot_general(
            xb, w1_ref[0].astype(jnp.bfloat16), (((1,), (1,)), ((), ())),
            preferred_element_type=jnp.float32,
        ) + b1_ref[0]
        h = h * 0.5 * (1.0 + lax.erf(h * 0.7071067811865476))
        y = lax.dot_general(
            h.astype(jnp.bfloat16), w2_ref[0].astype(jnp.bfloat16),
            (((1,), (1,)), ((), ())),
            preferred_element_type=jnp.float32,
        )
        out_ref[...] = y + b2_ref[0]


def _sc_gather_rows(table, idx, n_rows, ch):
    rows_per_w = n_rows // NW
    chunks = rows_per_w // ch
    mesh = plsc.VectorSubcoreMesh(core_axis_name="c", subcore_axis_name="s")

    @functools.partial(
        pl.kernel,
        out_type=jax.ShapeDtypeStruct((n_rows, H), jnp.float32),
        scratch_types=[
            pltpu.VMEM((ch,), jnp.int32),
            pltpu.VMEM((ch, H), jnp.float32),
            pltpu.SemaphoreType.DMA,
        ],
        mesh=mesh,
    )
    def k(table_hbm, idx_hbm, out_hbm, idx_v, rows_v, sem):
        wid = lax.axis_index("s") * 2 + lax.axis_index("c")
        for c in range(chunks):
            base = wid * rows_per_w + c * ch
            pltpu.sync_copy(idx_hbm.at[pl.ds(base, ch)], idx_v)
            pltpu.async_copy(table_hbm.at[idx_v], rows_v, sem).wait()
            pltpu.sync_copy(rows_v, out_hbm.at[pl.ds(base, ch)])

    return k(table, idx)


def kernel(hidden_states, W1, b1, W2, b2, Wg, bg):
    Bs, Ss, Hd = hidden_states.shape
    x = hidden_states.reshape(-1, Hd)

    slot2, be2, na2, lb2 = pl.pallas_call(
        _gate_body,
        out_shape=(
            jax.ShapeDtypeStruct((N, 1), jnp.int32),
            jax.ShapeDtypeStruct((NBLK, 1), jnp.int32),
            jax.ShapeDtypeStruct((1, 1), jnp.int32),
            jax.ShapeDtypeStruct((1, 1), jnp.float32),
        ),
    )(x, Wg, bg.reshape(1, E))
    slot = slot2[:, 0]
    be = be2[:, 0]
    na = na2[0]

    ar_n = jnp.arange(N, dtype=jnp.int32)
    gidx = (jnp.arange(NPAD, dtype=jnp.int32) % N).at[slot].set(ar_n)
    pos = slot

    x_sorted = _sc_gather_rows(x, gidx, NPAD, 128)

    grid_spec = pltpu.PrefetchScalarGridSpec(
        num_scalar_prefetch=2,
        grid=(NBLK,),
        in_specs=[
            pl.BlockSpec((BT, H), lambda b, be, na: (jnp.minimum(b, na[0] - 1), 0)),
            pl.BlockSpec((1, I, H), lambda b, be, na: (be[b], 0, 0)),
            pl.BlockSpec((1, 1, I), lambda b, be, na: (be[b], 0, 0)),
            pl.BlockSpec((1, H, I), lambda b, be, na: (be[b], 0, 0)),
            pl.BlockSpec((1, 1, H), lambda b, be, na: (be[b], 0, 0)),
        ],
        out_specs=pl.BlockSpec(
            (BT, H), lambda b, be, na: (jnp.minimum(b, na[0] - 1), 0)
        ),
    )
    y_sorted = pl.pallas_call(
        _ffn_body,
        grid_spec=grid_spec,
        out_shape=jax.ShapeDtypeStruct((NPAD, H), jnp.float32),
    )(be, na, x_sorted, W1, b1.reshape(E, 1, I), W2, b2.reshape(E, 1, H))

    out = _sc_gather_rows(y_sorted, pos, N, 128)
    return out.reshape(Bs, Ss, Hd), lb2[0, 0]


# speedup vs baseline: 1.1823x; 1.1087x over previous
import functools

import jax
import jax.numpy as jnp
from jax import lax
from jax.experimental import pallas as pl
from jax.experimental.pallas import tpu as pltpu
from jax.experimental.pallas import tpu_sc as plsc

H = 768
I = 3072
E = 64
N = 4096
BT = 128
NBLK = N // BT + E
NPAD = NBLK * BT
NW = 32
LB_COEF = 0.01


def _gate_body(x_ref, wg_ref, bg_ref, slot_ref, be_ref, na_ref, lb_ref):
    x = x_ref[...]
    wg = wg_ref[...]
    logits = lax.dot_general(
        x.astype(jnp.bfloat16), wg.astype(jnp.bfloat16), (((1,), (1,)), ((), ())),
        preferred_element_type=jnp.float32,
    ) + bg_ref[...]
    m = jnp.max(logits, axis=1, keepdims=True)
    col = lax.broadcasted_iota(jnp.int32, logits.shape, 1)
    eid = jnp.min(jnp.where(logits == m, col, E), axis=1, keepdims=True)
    oh = (col == eid).astype(jnp.int32)
    csum = oh
    sh = 1
    while sh < N:
        csum = csum + jnp.concatenate(
            [jnp.zeros((sh, E), jnp.int32), csum[: N - sh]], axis=0
        )
        sh *= 2
    counts = csum[N - 1 : N, :]
    rank = jnp.sum(oh * csum, axis=1, keepdims=True) - 1

    nb = (counts + BT - 1) // BT
    pb = nb * BT

    def lane_scan(v):
        s = 1
        while s < E:
            v = v + jnp.concatenate(
                [jnp.zeros((1, s), jnp.int32), v[:, : E - s]], axis=1
            )
            s *= 2
        return v

    cnb = lane_scan(nb)
    pstart = lane_scan(pb) - pb
    slot_ref[...] = jnp.sum(oh * pstart, axis=1, keepdims=True) + rank
    na = cnb[:, E - 1 : E]
    na_ref[...] = na
    rowb = lax.broadcasted_iota(jnp.int32, (NBLK, E), 0)
    ber = jnp.sum(
        (jnp.broadcast_to(cnb, (NBLK, E)) <= rowb).astype(jnp.int32),
        axis=1, keepdims=True,
    )
    bidx = lax.broadcasted_iota(jnp.int32, (NBLK, 1), 0)
    last_e = jnp.sum(jnp.where(bidx == na - 1, ber, 0), axis=0, keepdims=True)
    be_ref[...] = jnp.where(bidx < na, ber, last_e)
    mean = jnp.mean(logits, axis=0, keepdims=True)
    c = logits - mean
    var = jnp.sum(c * c, axis=0, keepdims=True) / (N - 1)
    ratio = var / (mean * mean + 1e-8)
    lb_ref[...] = LB_COEF * jnp.mean(ratio, axis=1, keepdims=True)


def _ffn_body(be_ref, na_ref, xs_ref, w1_ref, b1_ref, w2_ref, b2_ref, out_ref):
    b = pl.program_id(0)

    @pl.when(b < na_ref[0])
    def _():
        xb = xs_ref[...].astype(jnp.bfloat16)
        h = lax.dot_general(
            xb, w1_ref[0].astype(jnp.bfloat16), (((1,), (1,)), ((), ())),
            preferred_element_type=jnp.float32,
        ) + b1_ref[0]
        h = h * 0.5 * (1.0 + lax.erf(h * 0.7071067811865476))
        y = lax.dot_general(
            h.astype(jnp.bfloat16), w2_ref[0].astype(jnp.bfloat16),
            (((1,), (1,)), ((), ())),
            preferred_element_type=jnp.float32,
        )
        out_ref[...] = y + b2_ref[0]


def _sc_gather_rows(table, idx, n_rows, ch):
    rows_per_w = n_rows // NW
    chunks = rows_per_w // ch
    mesh = plsc.VectorSubcoreMesh(core_axis_name="c", subcore_axis_name="s")

    @functools.partial(
        pl.kernel,
        out_type=jax.ShapeDtypeStruct((n_rows, H), jnp.float32),
        scratch_types=[
            pltpu.VMEM((ch,), jnp.int32),
            pltpu.VMEM((ch, H), jnp.float32),
            pltpu.SemaphoreType.DMA,
        ],
        mesh=mesh,
    )
    def k(table_hbm, idx_hbm, out_hbm, idx_v, rows_v, sem):
        wid = lax.axis_index("s") * 2 + lax.axis_index("c")
        for c in range(chunks):
            base = wid * rows_per_w + c * ch
            pltpu.sync_copy(idx_hbm.at[pl.ds(base, ch)], idx_v)
            pltpu.async_copy(table_hbm.at[idx_v], rows_v, sem).wait()
            pltpu.sync_copy(rows_v, out_hbm.at[pl.ds(base, ch)])

    return k(table, idx)


def _sc_scatter_rows(x, slot):
    rows_per_w = N // NW
    mesh = plsc.VectorSubcoreMesh(core_axis_name="c", subcore_axis_name="s")

    @functools.partial(
        pl.kernel,
        out_type=jax.ShapeDtypeStruct((NPAD, H), jnp.float32),
        scratch_types=[
            pltpu.VMEM((rows_per_w,), jnp.int32),
            pltpu.VMEM((rows_per_w, H), jnp.float32),
            pltpu.SemaphoreType.DMA,
        ],
        mesh=mesh,
    )
    def k(x_hbm, slot_hbm, out_hbm, idx_v, rows_v, sem):
        wid = lax.axis_index("s") * 2 + lax.axis_index("c")
        base = wid * rows_per_w
        pltpu.sync_copy(slot_hbm.at[pl.ds(base, rows_per_w)], idx_v)
        pltpu.sync_copy(x_hbm.at[pl.ds(base, rows_per_w)], rows_v)
        pltpu.async_copy(rows_v, out_hbm.at[idx_v], sem).wait()

    return k(x, slot)


def kernel(hidden_states, W1, b1, W2, b2, Wg, bg):
    Bs, Ss, Hd = hidden_states.shape
    x = hidden_states.reshape(-1, Hd)

    slot2, be2, na2, lb2 = pl.pallas_call(
        _gate_body,
        out_shape=(
            jax.ShapeDtypeStruct((N, 1), jnp.int32),
            jax.ShapeDtypeStruct((NBLK, 1), jnp.int32),
            jax.ShapeDtypeStruct((1, 1), jnp.int32),
            jax.ShapeDtypeStruct((1, 1), jnp.float32),
        ),
    )(x, Wg, bg.reshape(1, E))
    slot = slot2[:, 0]
    be = be2[:, 0]
    na = na2[0]

    pos = slot
    x_sorted = _sc_scatter_rows(x, slot)

    grid_spec = pltpu.PrefetchScalarGridSpec(
        num_scalar_prefetch=2,
        grid=(NBLK,),
        in_specs=[
            pl.BlockSpec((BT, H), lambda b, be, na: (jnp.minimum(b, na[0] - 1), 0)),
            pl.BlockSpec((1, I, H), lambda b, be, na: (be[b], 0, 0)),
            pl.BlockSpec((1, 1, I), lambda b, be, na: (be[b], 0, 0)),
            pl.BlockSpec((1, H, I), lambda b, be, na: (be[b], 0, 0)),
            pl.BlockSpec((1, 1, H), lambda b, be, na: (be[b], 0, 0)),
        ],
        out_specs=pl.BlockSpec(
            (BT, H), lambda b, be, na: (jnp.minimum(b, na[0] - 1), 0)
        ),
    )
    y_sorted = pl.pallas_call(
        _ffn_body,
        grid_spec=grid_spec,
        out_shape=jax.ShapeDtypeStruct((NPAD, H), jnp.float32),
    )(be, na, x_sorted, W1, b1.reshape(E, 1, I), W2, b2.reshape(E, 1, H))

    out = _sc_gather_rows(y_sorted, pos, N, 128)
    return out.reshape(Bs, Ss, Hd), lb2[0, 0]
